# Initial kernel scaffold; baseline (speedup 1.0000x reference)
import jax
import jax.numpy as jnp
from jax.experimental import pallas as pl


def kernel(x, table, W, b):
    raise NotImplementedError("write your pallas kernel here")


# speedup vs baseline: 2.4079x; 2.4079x over previous
import functools

import jax
import jax.numpy as jnp
from jax import lax
from jax.experimental import pallas as pl
from jax.experimental.pallas import tpu as pltpu
from jax.experimental.pallas import tpu_sc as plsc

VOCAB = 1000000
EMBED = 32
NUM_CLASSES = 100
BATCH = 4096
HIST = 200

NC = 2
NS = 16
NW = NC * NS
B_PER_W = BATCH // NW
NBUF = 4
C0 = 128
C1 = HIST - C0
INV_HIST = 1.0 / HIST


def _sc_pool(x_hbm, table_hbm, out_hbm, idx_v, rows_v, pooled_v, sems):
  wid = lax.axis_index("s") * NC + lax.axis_index("c")
  base = wid * B_PER_W

  pltpu.sync_copy(x_hbm.at[pl.ds(base, B_PER_W), :], idx_v)

  def issue(row, s):
    pltpu.async_copy(
        table_hbm.at[idx_v.at[row, pl.ds(0, C0)]],
        rows_v.at[s, pl.ds(0, C0)], sems.at[s])
    pltpu.async_copy(
        table_hbm.at[idx_v.at[row, pl.ds(C0, C1)]],
        rows_v.at[s, pl.ds(C0, C1)], sems.at[s])

  def drain(s):
    pltpu.make_async_copy(
        table_hbm.at[pl.ds(0, HIST)], rows_v.at[s], sems.at[s]).wait()

  for s in range(NBUF):
    issue(s, s)

  @pl.loop(0, B_PER_W // NBUF)
  def _(g):
    for s in range(NBUF):
      row = g * NBUF + s
      drain(s)

      def red(j, carry):
        a0, a1 = carry
        a0 = a0 + rows_v[s, j, pl.ds(0, 16)]
        a1 = a1 + rows_v[s, j, pl.ds(16, 16)]
        return a0, a1

      zero = jnp.zeros((16,), jnp.float32)
      a0, a1 = lax.fori_loop(0, HIST, red, (zero, zero), unroll=8)

      @pl.when(row + NBUF < B_PER_W)
      def _():
        issue(row + NBUF, s)

      pooled_v[row, pl.ds(0, 16)] = a0 * INV_HIST
      pooled_v[row, pl.ds(16, 16)] = a1 * INV_HIST

  pltpu.sync_copy(pooled_v, out_hbm.at[pl.ds(base, B_PER_W), :])


@jax.jit
def _pooled_sc(x, table):
  mesh = plsc.VectorSubcoreMesh(
      core_axis_name="c", subcore_axis_name="s",
      num_cores=NC, num_subcores=NS)
  return pl.kernel(
      _sc_pool,
      out_type=jax.ShapeDtypeStruct((BATCH, EMBED), jnp.float32),
      mesh=mesh,
      compiler_params=pltpu.CompilerParams(use_tc_tiling_on_sc=False),
      scratch_types=[
          pltpu.VMEM((B_PER_W, HIST), jnp.int32),
          pltpu.VMEM((NBUF, HIST, EMBED), jnp.float32),
          pltpu.VMEM((B_PER_W, EMBED), jnp.float32),
          pltpu.SemaphoreType.DMA((NBUF,)),
      ],
  )(x, table)


def _linear_body(p_ref, wt_ref, b_ref, o_ref):
  o_ref[...] = jnp.dot(
      p_ref[...], wt_ref[...], preferred_element_type=jnp.float32
  ) + b_ref[...]


@jax.jit
def _linear_tc(pooled, Wt, b2):
  bm = 512
  return pl.pallas_call(
      _linear_body,
      grid=(BATCH // bm,),
      in_specs=[
          pl.BlockSpec((bm, EMBED), lambda i: (i, 0)),
          pl.BlockSpec((EMBED, NUM_CLASSES), lambda i: (0, 0)),
          pl.BlockSpec((1, NUM_CLASSES), lambda i: (0, 0)),
      ],
      out_specs=pl.BlockSpec((bm, NUM_CLASSES), lambda i: (i, 0)),
      out_shape=jax.ShapeDtypeStruct((BATCH, NUM_CLASSES), jnp.float32),
  )(pooled, Wt, b2)


def kernel(x, table, W, b):
  pooled = _pooled_sc(x.astype(jnp.int32), table)
  return _linear_tc(pooled, W.T, b.reshape(1, NUM_CLASSES))


# speedup vs baseline: 2.4094x; 1.0006x over previous
import functools

import jax
import jax.numpy as jnp
from jax import lax
from jax.experimental import pallas as pl
from jax.experimental.pallas import tpu as pltpu
from jax.experimental.pallas import tpu_sc as plsc

VOCAB = 1000000
EMBED = 32
NUM_CLASSES = 100
BATCH = 4096
HIST = 200

NC = 2
NS = 16
NW = NC * NS
B_PER_W = BATCH // NW
NBUF = 4
C0 = 128
C1 = HIST - C0
INV_HIST = 1.0 / HIST


def _sc_pool(x_hbm, table_hbm, out_hbm, idx_v, rows_v, pooled_v, sems):
  wid = lax.axis_index("s") * NC + lax.axis_index("c")
  base = wid * B_PER_W

  pltpu.sync_copy(x_hbm.at[pl.ds(base, B_PER_W), :], idx_v)

  def issue(row, s):
    pltpu.async_copy(
        table_hbm.at[idx_v.at[row, pl.ds(0, C0)]],
        rows_v.at[s, pl.ds(0, C0)], sems.at[s])
    pltpu.async_copy(
        table_hbm.at[idx_v.at[row, pl.ds(C0, C1)]],
        rows_v.at[s, pl.ds(C0, C1)], sems.at[s])

  def drain(s):
    pltpu.make_async_copy(
        table_hbm.at[pl.ds(0, HIST)], rows_v.at[s], sems.at[s]).wait()

  for s in range(NBUF):
    issue(s, s)

  @pl.loop(0, B_PER_W // NBUF)
  def _(g):
    for s in range(NBUF):
      row = g * NBUF + s
      drain(s)

      def red(j, carry):
        a0, a1 = carry
        a0 = a0 + rows_v[s, j, pl.ds(0, 16)]
        a1 = a1 + rows_v[s, j, pl.ds(16, 16)]
        return a0, a1

      zero = jnp.zeros((16,), jnp.float32)
      a0, a1 = lax.fori_loop(0, HIST, red, (zero, zero), unroll=8)

      @pl.when(row + NBUF < B_PER_W)
      def _():
        issue(row + NBUF, s)

      pooled_v[row, pl.ds(0, 16)] = a0 * INV_HIST
      pooled_v[row, pl.ds(16, 16)] = a1 * INV_HIST

  pltpu.sync_copy(pooled_v, out_hbm.at[pl.ds(base, B_PER_W), :])


@jax.jit
def _pooled_sc(x, table):
  mesh = plsc.VectorSubcoreMesh(
      core_axis_name="c", subcore_axis_name="s",
      num_cores=NC, num_subcores=NS)
  return pl.kernel(
      _sc_pool,
      out_type=jax.ShapeDtypeStruct((BATCH, EMBED), jnp.float32),
      mesh=mesh,
      compiler_params=pltpu.CompilerParams(use_tc_tiling_on_sc=False),
      scratch_types=[
          pltpu.VMEM((B_PER_W, HIST), jnp.int32),
          pltpu.VMEM((NBUF, HIST, EMBED), jnp.float32),
          pltpu.VMEM((B_PER_W, EMBED), jnp.float32),
          pltpu.SemaphoreType.DMA((NBUF,)),
      ],
  )(x, table)


def _xpose_body(t_ref, o_ref):
  t = t_ref[...].T
  o_ref[...] = jnp.concatenate([t[j::4, :] for j in range(4)], axis=1)


@jax.jit
def _linearize_tc(tT):
  bk = 8192
  return pl.pallas_call(
      _xpose_body,
      grid=(pl.cdiv(VOCAB, bk),),
      in_specs=[pl.BlockSpec((EMBED, bk), lambda i: (0, i))],
      out_specs=pl.BlockSpec((bk * EMBED // 128, 128), lambda i: (i, 0)),
      out_shape=jax.ShapeDtypeStruct((VOCAB * EMBED // 128, 128), jnp.float32),
  )(tT)


def _linear_body(p_ref, wt_ref, b_ref, o_ref):
  o_ref[...] = jnp.dot(
      p_ref[...], wt_ref[...], preferred_element_type=jnp.float32
  ) + b_ref[...]


@jax.jit
def _linear_tc(pooled, Wt, b2):
  bm = 512
  return pl.pallas_call(
      _linear_body,
      grid=(BATCH // bm,),
      in_specs=[
          pl.BlockSpec((bm, EMBED), lambda i: (i, 0)),
          pl.BlockSpec((EMBED, NUM_CLASSES), lambda i: (0, 0)),
          pl.BlockSpec((1, NUM_CLASSES), lambda i: (0, 0)),
      ],
      out_specs=pl.BlockSpec((bm, NUM_CLASSES), lambda i: (i, 0)),
      out_shape=jax.ShapeDtypeStruct((BATCH, NUM_CLASSES), jnp.float32),
  )(pooled, Wt, b2)


def kernel(x, table, W, b):
  t4 = jax.lax.optimization_barrier(table.reshape(VOCAB * EMBED // 128, 128))
  table_lin = t4.reshape(VOCAB, EMBED)
  pooled = _pooled_sc(x.astype(jnp.int32), table_lin)
  return _linear_tc(pooled, W.T, b.reshape(1, NUM_CLASSES))
